# baseline (speedup 1.0000x reference)
import functools

import jax
import jax.numpy as jnp
from jax import lax
from jax.experimental import pallas as pl
from jax.experimental.pallas import tpu as pltpu
from jax.experimental.pallas import tpu_sc as plsc

N = 10000
DEG = 32
D = 128
R = 8
OUT = 128

NC = 2
NS = 16
NW = NC * NS
NPW = 320
N_PAD = NW * NPW
E_PAD = N_PAD * DEG
CHUNK_E = 64
CN = CHUNK_E // DEG
NCH = (NPW * DEG) // CHUNK_E
NBUF = 2
NACC = NBUF
LANES = 16


def _scale_body(et_ref, sc_ref):
    et = et_ref[...]
    scale = jnp.zeros(et.shape, jnp.float32)
    for r in range(R):
        m = (et == r).astype(jnp.float32)
        cnt = jnp.sum(m, axis=1, keepdims=True)
        scale = scale + m / jnp.maximum(cnt, 1.0)
    bits = lax.bitcast_convert_type(scale, jnp.int32)
    bits = (bits & jnp.int32(~7)) | et
    sc_ref[...] = lax.bitcast_convert_type(bits, jnp.float32)


def _edge_meta(et2):
    BN = 2048
    return pl.pallas_call(
        _scale_body,
        grid=(N_PAD // BN,),
        in_specs=[pl.BlockSpec((BN, DEG), lambda i: (i, 0))],
        out_specs=pl.BlockSpec((BN, DEG), lambda i: (i, 0)),
        out_shape=jax.ShapeDtypeStruct((N_PAD, DEG), jnp.float32),
    )(et2)


_SC_MESH = plsc.VectorSubcoreMesh(core_axis_name="c", subcore_axis_name="s")


@functools.partial(
    pl.kernel,
    mesh=_SC_MESH,
    out_type=jax.ShapeDtypeStruct((N_PAD * R, OUT), jnp.float32),
    scratch_types=(
        [pltpu.VMEM((NPW * DEG,), jnp.int32)]
        + [pltpu.VMEM((NPW * DEG,), jnp.float32)]
        + [pltpu.VMEM((CHUNK_E, OUT), jnp.float32) for _ in range(NBUF)]
        + [pltpu.VMEM((CN * R, OUT), jnp.float32) for _ in range(NACC)]
        + [pltpu.VMEM_SHARED((N_PAD, OUT), jnp.float32)]
        + [pltpu.SemaphoreType.DMA for _ in range(NBUF)]
        + [pltpu.SemaphoreType.DMA for _ in range(NACC)]
    ),
)
def _sc_means(xt, colf, sclf, out, colv, sclv, *rest):
    rows = rest[:NBUF]
    accs = rest[NBUF:NBUF + NACC]
    xs = rest[NBUF + NACC]
    gsems = rest[NBUF + NACC + 1:NBUF + NACC + 1 + NBUF]
    wsems = rest[NBUF + NACC + 1 + NBUF:]
    cid = lax.axis_index("c")
    sid = lax.axis_index("s")
    wid = sid * NC + cid
    nbase = wid * NPW
    ebase = nbase * DEG

    pltpu.sync_copy(xt.at[pl.ds(sid * (N_PAD // NS), N_PAD // NS)],
                    xs.at[pl.ds(sid * (N_PAD // NS), N_PAD // NS)])
    plsc.subcore_barrier()
    pltpu.sync_copy(colf.at[pl.ds(ebase, NPW * DEG)], colv)
    pltpu.sync_copy(sclf.at[pl.ds(ebase, NPW * DEG)], sclv)

    def gather(c, b):
        return pltpu.make_async_copy(
            xs.at[colv.at[pl.ds(c * CHUNK_E, CHUNK_E)]], rows[b], gsems[b])

    def mwrite(c, a):
        return pltpu.make_async_copy(
            accs[a],
            out.at[pl.ds((nbase + c * CN) * R, CN * R)], wsems[a])

    for b in range(NBUF):
        gather(b, b).start()

    def chunk_group(g, carry):
        for b in range(NBUF):
            c = g * NBUF + b
            a_idx = b
            rows_b = rows[b]
            acc_b = accs[a_idx]

            @pl.when(c >= NACC)
            def _():
                mwrite(c - NACC, a_idx).wait()

            def zero_body(q, carry2):
                zero = jnp.zeros((LANES,), jnp.float32)
                for k in range(OUT // LANES):
                    acc_b[q, pl.ds(k * LANES, LANES)] = zero
                return carry2

            lax.fori_loop(0, CN * R, zero_body, 0)

            gather(c, b).wait()

            def node_body(n, carry2):
                for h in range(DEG // LANES):
                    sv = sclv[pl.ds(c * CHUNK_E + n * DEG + h * LANES, LANES)]
                    for j in range(LANES):
                        e = n * DEG + h * LANES + j
                        ts = sv[j]
                        t = lax.bitcast_convert_type(ts, jnp.int32) & 7
                        slot = n * R + t
                        s = jnp.full((LANES,), ts, jnp.float32)
                        for k in range(OUT // LANES):
                            val = s * rows_b[e, pl.ds(k * LANES, LANES)]
                            plsc.addupdate(
                                acc_b.at[slot, pl.ds(k * LANES, LANES)], val)
                return carry2

            lax.fori_loop(0, CN, node_body, 0)

            mwrite(c, a_idx).start()

            @pl.when(c + NBUF < NCH)
            def _():
                gather(c + NBUF, b).start()
        return carry

    lax.fori_loop(0, NCH // NBUF, chunk_group, 0)
    for a_idx in range(NACC):
        c_tail = NCH - NACC + a_idx
        mwrite(c_tail, c_tail % NBUF).wait()


def _final_body(m_ref, x_ref, wr_ref, wroot_ref, b_ref, o_ref):
    o_ref[...] = (
        jnp.dot(m_ref[...], wr_ref[...], preferred_element_type=jnp.float32)
        + jnp.dot(x_ref[...], wroot_ref[...],
                  preferred_element_type=jnp.float32)
        + b_ref[...])


def _final_matmul(m2, x_pad, wr, wroot, bias):
    BN = 512
    return pl.pallas_call(
        _final_body,
        grid=(N_PAD // BN,),
        in_specs=[
            pl.BlockSpec((BN, R * D), lambda i: (i, 0)),
            pl.BlockSpec((BN, D), lambda i: (i, 0)),
            pl.BlockSpec((R * D, OUT), lambda i: (0, 0)),
            pl.BlockSpec((D, OUT), lambda i: (0, 0)),
            pl.BlockSpec((OUT,), lambda i: (0,)),
        ],
        out_specs=pl.BlockSpec((BN, OUT), lambda i: (i, 0)),
        out_shape=jax.ShapeDtypeStruct((N_PAD, OUT), jnp.float32),
    )(m2, x_pad, wr, wroot, bias)


def kernel(x_feat, csr_row_ptr, csr_col_ind, edge_type, weight, bias):
    del csr_row_ptr
    x_pad = jnp.zeros((N_PAD, D), jnp.float32).at[:N].set(x_feat)
    et2 = jnp.zeros((N_PAD, DEG), jnp.int32).at[:N].set(
        edge_type.reshape(N, DEG))
    col2 = jnp.zeros((N_PAD, DEG), jnp.int32).at[:N].set(
        csr_col_ind.reshape(N, DEG))

    sclf = _edge_meta(et2).reshape(E_PAD)
    colf = col2.reshape(E_PAD)

    m = _sc_means(x_pad, colf, sclf)
    m2 = m.reshape(N_PAD, R * D)
    wr = weight[:R].reshape(R * D, OUT)
    y_pad = _final_matmul(m2, x_pad, wr, weight[R], bias)
    return y_pad[:N]


# speedup vs baseline: 1.0718x; 1.0718x over previous
import functools

import jax
import jax.numpy as jnp
from jax import lax
from jax.experimental import pallas as pl
from jax.experimental.pallas import tpu as pltpu
from jax.experimental.pallas import tpu_sc as plsc

N = 10000
DEG = 32
D = 128
R = 8
OUT = 128

NC = 2
NS = 16
NW = NC * NS
NPW = 320
N_PAD = NW * NPW
E_PAD = N_PAD * DEG
CHUNK_E = 64
CN = CHUNK_E // DEG
NCH = (NPW * DEG) // CHUNK_E
NBUF = 2
NACC = NBUF
LANES = 16


def _scale_body(et_ref, sc_ref):
    et = et_ref[...]
    parts = []
    for r in range(R):
        m = (et == r).astype(jnp.float32)
        cnt = jnp.sum(m, axis=1, keepdims=True)
        parts.append(m / jnp.maximum(cnt, 1.0))
    sc_ref[...] = jnp.concatenate(parts, axis=1)


def _edge_meta(et2):
    BN = 2048
    return pl.pallas_call(
        _scale_body,
        grid=(N_PAD // BN,),
        in_specs=[pl.BlockSpec((BN, DEG), lambda i: (i, 0))],
        out_specs=pl.BlockSpec((BN, R * DEG), lambda i: (i, 0)),
        out_shape=jax.ShapeDtypeStruct((N_PAD, R * DEG), jnp.float32),
    )(et2)


_SC_MESH = plsc.VectorSubcoreMesh(core_axis_name="c", subcore_axis_name="s")


@functools.partial(
    pl.kernel,
    mesh=_SC_MESH,
    out_type=jax.ShapeDtypeStruct((N_PAD * R, OUT), jnp.float32),
    scratch_types=(
        [pltpu.VMEM((NPW * DEG,), jnp.int32)]
        + [pltpu.VMEM((CHUNK_E, OUT), jnp.float32) for _ in range(NBUF)]
        + [pltpu.VMEM((R, CHUNK_E), jnp.float32) for _ in range(NBUF)]
        + [pltpu.VMEM((CN * R, OUT), jnp.float32) for _ in range(NACC)]
        + [pltpu.VMEM_SHARED((N_PAD, OUT), jnp.float32)]
        + [pltpu.SemaphoreType.DMA for _ in range(NBUF)]
        + [pltpu.SemaphoreType.DMA for _ in range(NBUF)]
        + [pltpu.SemaphoreType.DMA for _ in range(NACC)]
    ),
)
def _sc_means(xt, colf, smat, out, colv, *rest):
    rows = rest[:NBUF]
    smv = rest[NBUF:2 * NBUF]
    accs = rest[2 * NBUF:2 * NBUF + NACC]
    xs = rest[2 * NBUF + NACC]
    sems = rest[2 * NBUF + NACC + 1:]
    gsems = sems[:NBUF]
    ssems = sems[NBUF:2 * NBUF]
    wsems = sems[2 * NBUF:]
    cid = lax.axis_index("c")
    sid = lax.axis_index("s")
    wid = sid * NC + cid
    nbase = wid * NPW
    ebase = nbase * DEG

    pltpu.sync_copy(xt.at[pl.ds(sid * (N_PAD // NS), N_PAD // NS)],
                    xs.at[pl.ds(sid * (N_PAD // NS), N_PAD // NS)])
    plsc.subcore_barrier()
    pltpu.sync_copy(colf.at[pl.ds(ebase, NPW * DEG)], colv)

    def gather(c, b):
        return pltpu.make_async_copy(
            xs.at[colv.at[pl.ds(c * CHUNK_E, CHUNK_E)]], rows[b], gsems[b])

    def sload(c, b):
        return pltpu.make_async_copy(
            smat.at[wid * NCH + c], smv[b], ssems[b])

    def mwrite(c, a):
        return pltpu.make_async_copy(
            accs[a],
            out.at[pl.ds((nbase + c * CN) * R, CN * R)], wsems[a])

    for b in range(NBUF):
        gather(b, b).start()
        sload(b, b).start()

    def chunk_group(g, carry):
        for b in range(NBUF):
            c = g * NBUF + b
            rows_b = rows[b]
            smv_b = smv[b]
            acc_b = accs[b]

            @pl.when(c >= NACC)
            def _():
                mwrite(c - NACC, b).wait()

            gather(c, b).wait()
            sload(c, b).wait()

            def node_body(n, carry2):
                for q in range(2):
                    accv = tuple(jnp.zeros((LANES,), jnp.float32)
                                 for _ in range(R * 4))
                    for h in range(DEG // LANES):
                        svecs = [smv_b[r, pl.ds(n * DEG + h * LANES, LANES)]
                                 for r in range(R)]

                        def edge_body(j, accv_in, h=h, svecs=svecs):
                            e = n * DEG + h * LANES + j
                            jv = jnp.full((LANES, 1), j, jnp.int32)
                            dn = lax.GatherDimensionNumbers(
                                offset_dims=(), collapsed_slice_dims=(0,),
                                start_index_map=(0,))
                            sb = [lax.gather(
                                svecs[r], jv, dimension_numbers=dn,
                                slice_sizes=(1,),
                                mode=lax.GatherScatterMode.PROMISE_IN_BOUNDS)
                                for r in range(R)]
                            rowk = [rows_b[e, pl.ds((q * 4 + k) * LANES,
                                                    LANES)]
                                    for k in range(4)]
                            return tuple(
                                accv_in[r * 4 + k] + sb[r] * rowk[k]
                                for r in range(R) for k in range(4)
                            )

                        accv = lax.fori_loop(0, LANES, edge_body, accv)
                    for r in range(R):
                        for k in range(4):
                            acc_b[n * R + r,
                                  pl.ds((q * 4 + k) * LANES, LANES)] = (
                                accv[r * 4 + k])
                return carry2

            lax.fori_loop(0, CN, node_body, 0)

            mwrite(c, b).start()

            @pl.when(c + NBUF < NCH)
            def _():
                gather(c + NBUF, b).start()
                sload(c + NBUF, b).start()
        return carry

    lax.fori_loop(0, NCH // NBUF, chunk_group, 0)
    for a_idx in range(NACC):
        c_tail = NCH - NACC + a_idx
        mwrite(c_tail, c_tail % NBUF).wait()


def _final_body(m_ref, x_ref, wr_ref, wroot_ref, b_ref, o_ref):
    o_ref[...] = (
        jnp.dot(m_ref[...], wr_ref[...], preferred_element_type=jnp.float32)
        + jnp.dot(x_ref[...], wroot_ref[...],
                  preferred_element_type=jnp.float32)
        + b_ref[...])


def _final_matmul(m2, x_pad, wr, wroot, bias):
    BN = 512
    return pl.pallas_call(
        _final_body,
        grid=(N_PAD // BN,),
        in_specs=[
            pl.BlockSpec((BN, R * D), lambda i: (i, 0)),
            pl.BlockSpec((BN, D), lambda i: (i, 0)),
            pl.BlockSpec((R * D, OUT), lambda i: (0, 0)),
            pl.BlockSpec((D, OUT), lambda i: (0, 0)),
            pl.BlockSpec((OUT,), lambda i: (0,)),
        ],
        out_specs=pl.BlockSpec((BN, OUT), lambda i: (i, 0)),
        out_shape=jax.ShapeDtypeStruct((N_PAD, OUT), jnp.float32),
    )(m2, x_pad, wr, wroot, bias)


def kernel(x_feat, csr_row_ptr, csr_col_ind, edge_type, weight, bias):
    del csr_row_ptr
    x_pad = jnp.zeros((N_PAD, D), jnp.float32).at[:N].set(x_feat)
    et2 = jnp.zeros((N_PAD, DEG), jnp.int32).at[:N].set(
        edge_type.reshape(N, DEG))
    col2 = jnp.zeros((N_PAD, DEG), jnp.int32).at[:N].set(
        csr_col_ind.reshape(N, DEG))

    sc2 = _edge_meta(et2)
    smat = (sc2.reshape(N_PAD // CN, CN, R, DEG)
            .transpose(0, 2, 1, 3)
            .reshape(N_PAD // CN, R, CHUNK_E))
    colf = col2.reshape(E_PAD)

    m = _sc_means(x_pad, colf, smat)
    m2 = m.reshape(N_PAD, R * D)
    wr = weight[:R].reshape(R * D, OUT)
    y_pad = _final_matmul(m2, x_pad, wr, weight[R], bias)
    return y_pad[:N]
